# Initial kernel scaffold; baseline (speedup 1.0000x reference)
import jax
import jax.numpy as jnp
from jax.experimental import pallas as pl


def kernel(x, params):
    raise NotImplementedError("write your pallas kernel here")


# speedup vs baseline: 12.7845x; 12.7845x over previous
import functools

import jax
import jax.numpy as jnp
from jax import lax
from jax.experimental import pallas as pl
from jax.experimental.pallas import tpu as pltpu
from jax.experimental.pallas import tpu_sc as plsc

_RADIUS = (0.1, 0.2, 0.4)
_NSAMPLES = (16, 32, 128)
_NPOINT = 512
_B, _N, _C = 8, 4096, 3
_BS = _B * _NPOINT
_MT = 512
_EPS = 1e-5


def _fps_body(xx_ref, xy_ref, xz_ref, cx_ref, cy_ref, cz_ref):
    xx = xx_ref[...]
    xy = xy_ref[...]
    xz = xz_ref[...]
    iota = lax.broadcasted_iota(jnp.int32, (_B, _N), 1)
    iota_s = lax.broadcasted_iota(jnp.int32, (_B, _NPOINT), 1)

    def step(i, carry):
        dist, far, acx, acy, acz = carry
        onehot = iota == far
        cx = jnp.sum(jnp.where(onehot, xx, 0.0), axis=1, keepdims=True)
        cy = jnp.sum(jnp.where(onehot, xy, 0.0), axis=1, keepdims=True)
        cz = jnp.sum(jnp.where(onehot, xz, 0.0), axis=1, keepdims=True)
        sel = iota_s == i
        acx = jnp.where(sel, cx, acx)
        acy = jnp.where(sel, cy, acy)
        acz = jnp.where(sel, cz, acz)
        d = (xx - cx) ** 2 + (xy - cy) ** 2 + (xz - cz) ** 2
        dist = jnp.minimum(dist, d)
        maxv = jnp.max(dist, axis=1, keepdims=True)
        idxe = jnp.where(dist == maxv, iota, _N)
        far = jnp.min(idxe, axis=1, keepdims=True)
        return dist, far, acx, acy, acz

    dist0 = jnp.full((_B, _N), 1e10, dtype=jnp.float32)
    far0 = jnp.zeros((_B, 1), dtype=jnp.int32)
    zs = jnp.zeros((_B, _NPOINT), dtype=jnp.float32)
    _, _, acx, acy, acz = lax.fori_loop(0, _NPOINT, step,
                                        (dist0, far0, zs, zs, zs))
    cx_ref[...] = acx
    cy_ref[...] = acy
    cz_ref[...] = acz


def _run_fps(xx, xy, xz):
    out = jax.ShapeDtypeStruct((_B, _NPOINT), jnp.float32)
    return pl.pallas_call(
        _fps_body,
        out_shape=(out, out, out),
    )(xx, xy, xz)


def _bn_params(w, b, g, be, mu_in, sig_in):
    mu = jnp.dot(w, mu_in, preferred_element_type=jnp.float32) + b
    q = jnp.sum(jnp.dot(w, sig_in, preferred_element_type=jnp.float32) * w,
                axis=1, keepdims=True)
    ez2 = q + 2.0 * b * mu - b * b
    var = ez2 - mu * mu
    s = g * lax.rsqrt(var + _EPS)
    t = be - s * mu
    return s, t


def _p1_body(k_count, gx_ref, sg1_ref, sg2_ref):
    @pl.when(pl.program_id(0) == 0)
    def _():
        sg1_ref[...] = jnp.zeros_like(sg1_ref)
        sg2_ref[...] = jnp.zeros_like(sg2_ref)

    a1 = jnp.zeros((3, 1), jnp.float32)
    a2 = jnp.zeros((3, 3), jnp.float32)
    for k in range(k_count):
        G = gx_ref[:, k, :]
        a1 = a1 + jnp.sum(G, axis=1, keepdims=True)
        a2 = a2 + lax.dot_general(G, G, (((1,), (1,)), ((), ())),
                                  preferred_element_type=jnp.float32)
    sg1_ref[...] += a1
    sg2_ref[...] += a2


def _p2_body(k_count, n_elems, gx_ref, sg1_ref, sg2_ref,
             w1_ref, b1_ref, g1_ref, e1_ref, sa1_ref, sa2_ref):
    c1 = w1_ref.shape[0]
    s1, t1 = _bn_params(w1_ref[...], b1_ref[...], g1_ref[...], e1_ref[...],
                        sg1_ref[...] / n_elems, sg2_ref[...] / n_elems)

    @pl.when(pl.program_id(0) == 0)
    def _():
        sa1_ref[...] = jnp.zeros_like(sa1_ref)
        sa2_ref[...] = jnp.zeros_like(sa2_ref)

    a1 = jnp.zeros((c1, 1), jnp.float32)
    a2 = jnp.zeros((c1, c1), jnp.float32)
    w1 = w1_ref[...]
    for k in range(k_count):
        G = gx_ref[:, k, :]
        A = jnp.maximum(
            s1 * jnp.dot(w1, G, preferred_element_type=jnp.float32) + t1, 0.0)
        a1 = a1 + jnp.sum(A, axis=1, keepdims=True)
        a2 = a2 + lax.dot_general(A, A, (((1,), (1,)), ((), ())),
                                  preferred_element_type=jnp.float32)
    sa1_ref[...] += a1
    sa2_ref[...] += a2


def _p3_body(k_count, n_elems, gx_ref, sg1_ref, sg2_ref,
             w1_ref, b1_ref, g1_ref, e1_ref,
             w2_ref, b2_ref, g2_ref, e2_ref,
             sa1_ref, sa2_ref, sb1_ref, sb2_ref):
    c2 = w2_ref.shape[0]
    s1, t1 = _bn_params(w1_ref[...], b1_ref[...], g1_ref[...], e1_ref[...],
                        sg1_ref[...] / n_elems, sg2_ref[...] / n_elems)
    s2, t2 = _bn_params(w2_ref[...], b2_ref[...], g2_ref[...], e2_ref[...],
                        sa1_ref[...] / n_elems, sa2_ref[...] / n_elems)

    @pl.when(pl.program_id(0) == 0)
    def _():
        sb1_ref[...] = jnp.zeros_like(sb1_ref)
        sb2_ref[...] = jnp.zeros_like(sb2_ref)

    b1acc = jnp.zeros((c2, 1), jnp.float32)
    b2acc = jnp.zeros((c2, c2), jnp.float32)
    w1 = w1_ref[...]
    w2 = w2_ref[...]
    for k in range(k_count):
        G = gx_ref[:, k, :]
        A1 = jnp.maximum(
            s1 * jnp.dot(w1, G, preferred_element_type=jnp.float32) + t1, 0.0)
        A2 = jnp.maximum(
            s2 * jnp.dot(w2, A1, preferred_element_type=jnp.float32) + t2, 0.0)
        b1acc = b1acc + jnp.sum(A2, axis=1, keepdims=True)
        b2acc = b2acc + lax.dot_general(A2, A2, (((1,), (1,)), ((), ())),
                                        preferred_element_type=jnp.float32)
    sb1_ref[...] += b1acc
    sb2_ref[...] += b2acc


def _p4_body(k_count, n_elems, gx_ref, sg1_ref, sg2_ref,
             w1_ref, b1_ref, g1_ref, e1_ref,
             w2_ref, b2_ref, g2_ref, e2_ref,
             w3_ref, b3_ref, g3_ref, e3_ref,
             sa1_ref, sa2_ref, sb1_ref, sb2_ref, out_ref):
    c3 = w3_ref.shape[0]
    s1, t1 = _bn_params(w1_ref[...], b1_ref[...], g1_ref[...], e1_ref[...],
                        sg1_ref[...] / n_elems, sg2_ref[...] / n_elems)
    s2, t2 = _bn_params(w2_ref[...], b2_ref[...], g2_ref[...], e2_ref[...],
                        sa1_ref[...] / n_elems, sa2_ref[...] / n_elems)
    s3, t3 = _bn_params(w3_ref[...], b3_ref[...], g3_ref[...], e3_ref[...],
                        sb1_ref[...] / n_elems, sb2_ref[...] / n_elems)

    w1 = w1_ref[...]
    w2 = w2_ref[...]
    w3 = w3_ref[...]
    acc = jnp.full((c3, _MT), -jnp.inf, jnp.float32)
    for k in range(k_count):
        G = gx_ref[:, k, :]
        A1 = jnp.maximum(
            s1 * jnp.dot(w1, G, preferred_element_type=jnp.float32) + t1, 0.0)
        A2 = jnp.maximum(
            s2 * jnp.dot(w2, A1, preferred_element_type=jnp.float32) + t2, 0.0)
        A3 = jnp.maximum(
            s3 * jnp.dot(w3, A2, preferred_element_type=jnp.float32) + t3, 0.0)
        acc = jnp.maximum(acc, A3)
    out_ref[...] = acc


def _small_spec(shape):
    return pl.BlockSpec(shape, lambda m: tuple(0 for _ in shape))


def _run_branch(gx, layers, k_count):
    n_elems = float(_B * _NPOINT * k_count)
    grid = (_BS // _MT,)
    gx_spec = pl.BlockSpec((3, k_count, _MT), lambda m: (0, 0, m))

    (w1, b1, g1, e1), (w2, b2, g2, e2), (w3, b3, g3, e3) = layers
    c1, c2, c3 = w1.shape[0], w2.shape[0], w3.shape[0]

    sg1, sg2 = pl.pallas_call(
        functools.partial(_p1_body, k_count),
        grid=grid,
        in_specs=[gx_spec],
        out_specs=(_small_spec((3, 1)), _small_spec((3, 3))),
        out_shape=(jax.ShapeDtypeStruct((3, 1), jnp.float32),
                   jax.ShapeDtypeStruct((3, 3), jnp.float32)),
    )(gx)

    l1_specs = [_small_spec((c1, 3)), _small_spec((c1, 1)),
                _small_spec((c1, 1)), _small_spec((c1, 1))]
    l2_specs = [_small_spec((c2, c1)), _small_spec((c2, 1)),
                _small_spec((c2, 1)), _small_spec((c2, 1))]
    l3_specs = [_small_spec((c3, c2)), _small_spec((c3, 1)),
                _small_spec((c3, 1)), _small_spec((c3, 1))]

    sa1, sa2 = pl.pallas_call(
        functools.partial(_p2_body, k_count, n_elems),
        grid=grid,
        in_specs=[gx_spec, _small_spec((3, 1)), _small_spec((3, 3))] + l1_specs,
        out_specs=(_small_spec((c1, 1)), _small_spec((c1, c1))),
        out_shape=(jax.ShapeDtypeStruct((c1, 1), jnp.float32),
                   jax.ShapeDtypeStruct((c1, c1), jnp.float32)),
    )(gx, sg1, sg2, w1, b1, g1, e1)

    sb1, sb2 = pl.pallas_call(
        functools.partial(_p3_body, k_count, n_elems),
        grid=grid,
        in_specs=([gx_spec, _small_spec((3, 1)), _small_spec((3, 3))]
                  + l1_specs + l2_specs
                  + [_small_spec((c1, 1)), _small_spec((c1, c1))]),
        out_specs=(_small_spec((c2, 1)), _small_spec((c2, c2))),
        out_shape=(jax.ShapeDtypeStruct((c2, 1), jnp.float32),
                   jax.ShapeDtypeStruct((c2, c2), jnp.float32)),
    )(gx, sg1, sg2, w1, b1, g1, e1, w2, b2, g2, e2, sa1, sa2)

    out = pl.pallas_call(
        functools.partial(_p4_body, k_count, n_elems),
        grid=grid,
        in_specs=([gx_spec, _small_spec((3, 1)), _small_spec((3, 3))]
                  + l1_specs + l2_specs + l3_specs
                  + [_small_spec((c1, 1)), _small_spec((c1, c1)),
                     _small_spec((c2, 1)), _small_spec((c2, c2))]),
        out_specs=pl.BlockSpec((c3, _MT), lambda m: (0, m)),
        out_shape=jax.ShapeDtypeStruct((c3, _BS), jnp.float32),
    )(gx, sg1, sg2, w1, b1, g1, e1, w2, b2, g2, e2, w3, b3, g3, e3,
      sa1, sa2, sb1, sb2)
    return out


_ROWS_PER_TILE = _BS // 32
_NCHUNK = _N // 16


def _sc_group_body(xs_ref, cs_ref, g0_ref, g1_ref, g2_ref,
                   xloc, cloc, ib0, ib1, ib2, gl0, gl1, gl2):
    wid = lax.axis_index("s") * 2 + lax.axis_index("c")
    b = wid // 4
    q = wid % 4

    for c3 in range(3):
        pltpu.sync_copy(xs_ref.at[pl.ds((b * 3 + c3) * _N, _N)],
                        xloc.at[pl.ds(c3 * _N, _N)])
        pltpu.sync_copy(
            cs_ref.at[pl.ds((b * 3 + c3) * _NPOINT + q * _ROWS_PER_TILE,
                            _ROWS_PER_TILE)],
            cloc.at[pl.ds(c3 * _ROWS_PER_TILE, _ROWS_PER_TILE)])

    iota16 = lax.iota(jnp.int32, 16)
    r2s = [r * r for r in _RADIUS]
    branches = ((ib0, gl0, _NSAMPLES[0], r2s[0]),
                (ib1, gl1, _NSAMPLES[1], r2s[1]),
                (ib2, gl2, _NSAMPLES[2], r2s[2]))

    def row_step(r, _):
        cvals = tuple(
            plsc.load_gather(
                cloc, [jnp.full((16,), c3 * _ROWS_PER_TILE, jnp.int32) + r])[0]
            for c3 in range(3))
        cxs, cys, czs = cvals

        def scan_chunk(c, carry):
            p0, p1, p2 = carry
            off = c * 16
            dx = xloc[pl.ds(pl.multiple_of(off, 16), 16)] - cxs
            dy = xloc[pl.ds(pl.multiple_of(_N + off, 16), 16)] - cys
            dz = xloc[pl.ds(pl.multiple_of(2 * _N + off, 16), 16)] - czs
            d = dx * dx + dy * dy + dz * dz
            gi = off + iota16
            newp = []
            for (ib, _gl, kc, r2), p in zip(branches, (p0, p1, p2)):
                m = d <= r2
                inc = plsc.cumsum(jnp.where(m, 1, 0))
                cnt = inc[15]

                @pl.when(p < kc)
                def _():
                    pos = p + inc - 1
                    plsc.store_scatter(ib, [pos], gi, mask=m & (pos < kc))

                newp.append(jnp.where(p < kc, p + cnt, p))
            return tuple(newp)

        p0, p1, p2 = lax.fori_loop(0, _NCHUNK, scan_chunk, (0, 0, 0))

        for (ib, gl, kc, _r2), p in zip(branches, (p0, p1, p2)):
            count = jnp.minimum(p, kc)
            first = ib[pl.ds(0, 16)][0]
            for ch in range(kc // 16):
                iv = ib[pl.ds(ch * 16, 16)]
                pos = ch * 16 + iota16
                ivp = jnp.where(pos < count, iv, first)
                tgt = pos * _ROWS_PER_TILE + r
                for c3 in range(3):
                    vals = plsc.load_gather(xloc, [ivp + c3 * _N]) - cvals[c3]
                    plsc.store_scatter(
                        gl, [tgt + c3 * (kc * _ROWS_PER_TILE)], vals)
        return 0

    lax.fori_loop(0, _ROWS_PER_TILE, row_step, 0)

    for (ib, gl, kc, _r2), gout in zip(branches, (g0_ref, g1_ref, g2_ref)):
        sz = 3 * kc * _ROWS_PER_TILE
        pltpu.sync_copy(gl, gout.at[pl.ds(wid * sz, sz)])


def _group_all(xs, cs):
    mesh = plsc.VectorSubcoreMesh(core_axis_name="c", subcore_axis_name="s")
    out_type = tuple(
        jax.ShapeDtypeStruct((32 * 3 * k * _ROWS_PER_TILE,), jnp.float32)
        for k in _NSAMPLES)
    scratch = [
        pltpu.VMEM((3 * _N,), jnp.float32),
        pltpu.VMEM((3 * _ROWS_PER_TILE,), jnp.float32),
        pltpu.VMEM((_NSAMPLES[0],), jnp.int32),
        pltpu.VMEM((_NSAMPLES[1],), jnp.int32),
        pltpu.VMEM((_NSAMPLES[2],), jnp.int32),
        pltpu.VMEM((3 * _NSAMPLES[0] * _ROWS_PER_TILE,), jnp.float32),
        pltpu.VMEM((3 * _NSAMPLES[1] * _ROWS_PER_TILE,), jnp.float32),
        pltpu.VMEM((3 * _NSAMPLES[2] * _ROWS_PER_TILE,), jnp.float32),
    ]
    fn = pl.kernel(_sc_group_body, out_type=out_type, mesh=mesh,
                   scratch_types=scratch,
                   compiler_params=pltpu.CompilerParams(
                       use_tc_tiling_on_sc=False,
                       needs_layout_passes=False))
    raw = fn(xs.reshape(-1), cs.reshape(-1))
    outs = []
    for arr, k in zip(raw, _NSAMPLES):
        gx = jnp.transpose(arr.reshape(32, 3, k, _ROWS_PER_TILE),
                           (1, 2, 0, 3)).reshape(3, k, _BS)
        outs.append(gx)
    return outs


def kernel(x, params):
    xx = x[:, :, 0]
    xy = x[:, :, 1]
    xz = x[:, :, 2]
    cx, cy, cz = _run_fps(xx, xy, xz)
    new_xyz = jnp.stack([cx, cy, cz], axis=-1)

    xs = jnp.stack([xx, xy, xz], axis=1)
    cs = jnp.stack([cx, cy, cz], axis=1)
    gxs = _group_all(xs, cs)

    feats = []
    for i, k_count in enumerate(_NSAMPLES):
        layers = [(l["W"], l["b"].reshape(-1, 1), l["g"].reshape(-1, 1),
                   l["beta"].reshape(-1, 1)) for l in params[i]]
        feats.append(_run_branch(gxs[i], layers, k_count))

    cat = jnp.concatenate(feats, axis=0)
    points = jnp.transpose(cat.reshape(-1, _B, _NPOINT), (1, 2, 0))
    return new_xyz, points
